# baseline (speedup 1.0000x reference)
import jax
import jax.numpy as jnp
from jax.experimental import pallas as pl
from jax.experimental.pallas import tpu as pltpu

_B, _N, _D, _H, _C = 16, 1024, 128, 128, 10
_DA = _D + 8
_CP = 128


def _fused_body(adj_ref, xt_ref, wrt_ref, wnt_ref, wca_ref, out_ref):
    A = (adj_ref[0] > 0.5).astype(jnp.bfloat16)
    xt = xt_ref[0]
    aggT = jax.lax.dot_general(
        xt[:_D], A,
        dimension_numbers=(((1,), (0,)), ((), ())),
        preferred_element_type=jnp.float32)
    rootT = jax.lax.dot_general(
        wrt_ref[...], xt,
        dimension_numbers=(((1,), (0,)), ((), ())),
        preferred_element_type=jnp.float32)
    nbrT = jax.lax.dot_general(
        wnt_ref[...], aggT.astype(jnp.bfloat16),
        dimension_numbers=(((1,), (0,)), ((), ())),
        preferred_element_type=jnp.float32)
    hT = jnp.maximum(rootT + nbrT, 0.0)
    pooledT = jnp.sum(hT, axis=1, keepdims=True) * (1.0 / _N)
    pooled = jnp.transpose(pooledT)
    out_ref[0] = jax.lax.dot_general(
        pooled, wca_ref[...],
        dimension_numbers=(((1,), (0,)), ((), ())),
        preferred_element_type=jnp.float32)


def kernel(x, adj, W_root, W_nbr, b, W_cls, b_cls):
    xt = jnp.swapaxes(x, 1, 2)
    ones = jnp.ones((_B, 1, _N), x.dtype)
    pad = jnp.zeros((_B, _DA - _D - 1, _N), x.dtype)
    xta = jnp.concatenate([xt, ones, pad], axis=1).astype(jnp.bfloat16)
    wrt = jnp.concatenate(
        [W_root.T, b[:, None], jnp.zeros((_H, _DA - _D - 1), jnp.float32)],
        axis=1).astype(jnp.bfloat16)
    wnt = W_nbr.T.astype(jnp.bfloat16)
    wca = jnp.zeros((_H, _CP), jnp.float32).at[:, :_C].set(W_cls)
    bca = jnp.zeros((_CP,), jnp.float32).at[:_C].set(b_cls)

    out = pl.pallas_call(
        _fused_body,
        grid=(_B,),
        in_specs=[
            pl.BlockSpec((1, _N, _N), lambda i: (i, 0, 0)),
            pl.BlockSpec((1, _DA, _N), lambda i: (i, 0, 0)),
            pl.BlockSpec((_H, _DA), lambda i: (0, 0)),
            pl.BlockSpec((_H, _D), lambda i: (0, 0)),
            pl.BlockSpec((_H, _CP), lambda i: (0, 0)),
        ],
        out_specs=pl.BlockSpec((1, 1, _CP), lambda i: (i, 0, 0)),
        out_shape=jax.ShapeDtypeStruct((_B, 1, _CP), jnp.float32),
        compiler_params=pltpu.CompilerParams(
            dimension_semantics=("arbitrary",)),
    )(adj, xta, wrt, wnt, wca)
    return out[:, 0, :_C] + bca[:_C]


# speedup vs baseline: 1.0143x; 1.0143x over previous
import jax
import jax.numpy as jnp
from jax.experimental import pallas as pl
from jax.experimental.pallas import tpu as pltpu

_B, _N, _D, _H, _C = 16, 1024, 128, 128, 10
_CP = 128


def _fused_body(adj_ref, x_ref, wcomb_ref, b_ref, wc_ref, bc_ref, out_ref):
    A = (adj_ref[0] > 0.5).astype(jnp.bfloat16)
    xh = x_ref[0]
    agg = jax.lax.dot_general(
        A, xh,
        dimension_numbers=(((0,), (0,)), ((), ())),
        preferred_element_type=jnp.float32)
    acts = jnp.concatenate([xh, agg.astype(jnp.bfloat16)], axis=1)
    h = jax.lax.dot_general(
        acts, wcomb_ref[...],
        dimension_numbers=(((1,), (0,)), ((), ())),
        preferred_element_type=jnp.float32)
    h = jnp.maximum(h + b_ref[...], 0.0)
    pooled = jnp.sum(h, axis=0, keepdims=True) * (1.0 / _N)
    out_ref[0] = jnp.dot(pooled, wc_ref[...],
                         preferred_element_type=jnp.float32) + bc_ref[...]


def kernel(x, adj, W_root, W_nbr, b, W_cls, b_cls):
    xh = x.astype(jnp.bfloat16)
    wcomb = jnp.concatenate([W_root, W_nbr], axis=0).astype(jnp.bfloat16)
    b2 = b.reshape(1, _H)
    wc = jnp.zeros((_H, _CP), jnp.float32).at[:, :_C].set(W_cls)
    bc = jnp.zeros((1, _CP), jnp.float32).at[0, :_C].set(b_cls)

    out = pl.pallas_call(
        _fused_body,
        grid=(_B,),
        in_specs=[
            pl.BlockSpec((1, _N, _N), lambda i: (i, 0, 0)),
            pl.BlockSpec((1, _N, _D), lambda i: (i, 0, 0)),
            pl.BlockSpec((2 * _D, _H), lambda i: (0, 0)),
            pl.BlockSpec((1, _H), lambda i: (0, 0)),
            pl.BlockSpec((_H, _CP), lambda i: (0, 0)),
            pl.BlockSpec((1, _CP), lambda i: (0, 0)),
        ],
        out_specs=pl.BlockSpec((1, 1, _CP), lambda i: (i, 0, 0)),
        out_shape=jax.ShapeDtypeStruct((_B, 1, _CP), jnp.float32),
        compiler_params=pltpu.CompilerParams(
            dimension_semantics=("arbitrary",)),
    )(adj, xh, wcomb, b2, wc, bc)
    return out[:, 0, :_C]
